# Initial kernel scaffold; baseline (speedup 1.0000x reference)
import jax
import jax.numpy as jnp
from jax.experimental import pallas as pl


def kernel(x, pos_emb_weight):
    raise NotImplementedError("write your pallas kernel here")


# speedup vs baseline: 1.5443x; 1.5443x over previous
import jax
import jax.numpy as jnp
from jax.experimental import pallas as pl

BS = 512


def _body(x_ref, pos_ref, out_ref):
    out_ref[...] = x_ref[...] + pos_ref[...][None]


def kernel(x, pos_emb_weight):
    batch, seq_len, d_model = x.shape
    grid = (seq_len // BS, batch)
    return pl.pallas_call(
        _body,
        grid=grid,
        in_specs=[
            pl.BlockSpec((1, BS, d_model), lambda s, b: (b, s, 0)),
            pl.BlockSpec((BS, d_model), lambda s, b: (s, 0)),
        ],
        out_specs=pl.BlockSpec((1, BS, d_model), lambda s, b: (b, s, 0)),
        out_shape=jax.ShapeDtypeStruct(x.shape, x.dtype),
    )(x, pos_emb_weight)
